# baseline (speedup 1.0000x reference)
import jax
import jax.numpy as jnp
from jax import lax
from jax.experimental import pallas as pl
from jax.experimental.pallas import tpu as pltpu
from jax.experimental.pallas import tpu_sc as plsc

N = 10000
F_IN = 256
HC = 64
NPAD = 10112
STRIPE = NPAD // 16
E = 160000
NC, NS = 2, 16
NW = NC * NS
EPW = 5376
EPAD = EPW * NW
CH = 128
NCHUNK = EPW // CH
NBUF = 3
BM = 400


def _leaky(x):
    return jnp.where(x >= 0, x, 0.2 * x)


def _dense1_body(x_ref, w_ref, asm_ref, adm_ref, h_ref, as_ref, ad_ref, mx_ref):
    i = pl.program_id(0)
    h = jnp.dot(x_ref[...], w_ref[...], preferred_element_type=jnp.float32)
    a_s = jnp.dot(h, asm_ref[...], preferred_element_type=jnp.float32)
    a_d = jnp.dot(h, adm_ref[...], preferred_element_type=jnp.float32)
    h_ref[...] = h
    as_ref[...] = a_s
    ad_ref[...] = a_d

    @pl.when(i == 0)
    def _():
        mx_ref[...] = jnp.full((2, 16), -3.0e38, jnp.float32)

    upd = jnp.concatenate(
        [jnp.max(a_s, axis=0, keepdims=True), jnp.max(a_d, axis=0, keepdims=True)],
        axis=0,
    )
    mx_ref[...] = jnp.maximum(mx_ref[...], upd)


def _dense1(x, w1, asm, adm):
    return pl.pallas_call(
        _dense1_body,
        grid=(N // BM,),
        in_specs=[
            pl.BlockSpec((BM, F_IN), lambda i: (i, 0)),
            pl.BlockSpec((F_IN, HC), lambda i: (0, 0)),
            pl.BlockSpec((HC, 16), lambda i: (0, 0)),
            pl.BlockSpec((HC, 16), lambda i: (0, 0)),
        ],
        out_specs=[
            pl.BlockSpec((BM, HC), lambda i: (i, 0)),
            pl.BlockSpec((BM, 16), lambda i: (i, 0)),
            pl.BlockSpec((BM, 16), lambda i: (i, 0)),
            pl.BlockSpec((2, 16), lambda i: (0, 0)),
        ],
        out_shape=[
            jax.ShapeDtypeStruct((N, HC), jnp.float32),
            jax.ShapeDtypeStruct((N, 16), jnp.float32),
            jax.ShapeDtypeStruct((N, 16), jnp.float32),
            jax.ShapeDtypeStruct((2, 16), jnp.float32),
        ],
    )(x, w1, asm, adm)


def _make_edge_kernel():
    mesh = plsc.VectorSubcoreMesh(core_axis_name="c", subcore_axis_name="s",
                                  num_cores=NC, num_subcores=NS)

    def body(h_hbm, as_hbm, ad_hbm, src_hbm, dst_hbm, kv_hbm, z64_hbm, z16_hbm,
             m_out, e_out, sidx, didx, sdix, hrows, arows, drows, orows,
             erows, kv_v, acc_m, acc_e, semg, sems):
        c = lax.axis_index("c")
        s = lax.axis_index("s")
        wid = s * NC + c
        pltpu.sync_copy(z64_hbm.at[pl.ds(s * STRIPE, STRIPE)],
                        acc_m.at[pl.ds(s * STRIPE, STRIPE)])
        pltpu.sync_copy(z16_hbm.at[pl.ds(s * STRIPE, STRIPE)],
                        acc_e.at[pl.ds(s * STRIPE, STRIPE)])
        pltpu.sync_copy(kv_hbm, kv_v)
        plsc.subcore_barrier()

        kv = kv_v[...]

        def issue_gathers(k, b):
            base = wid * EPW + k * CH
            pltpu.sync_copy(src_hbm.at[pl.ds(base, CH)], sidx.at[b])
            pltpu.sync_copy(dst_hbm.at[pl.ds(base, CH)], didx.at[b])
            pltpu.async_copy(h_hbm.at[sidx.at[b]], hrows.at[b], semg.at[b])
            pltpu.async_copy(as_hbm.at[sidx.at[b]], arows.at[b], semg.at[b])
            pltpu.async_copy(ad_hbm.at[didx.at[b]], drows.at[b], semg.at[b])

        def wait_gathers(b):
            pltpu.make_async_copy(h_hbm.at[sidx.at[b]], hrows.at[b],
                                  semg.at[b]).wait()
            pltpu.make_async_copy(as_hbm.at[sidx.at[b]], arows.at[b],
                                  semg.at[b]).wait()
            pltpu.make_async_copy(ad_hbm.at[didx.at[b]], drows.at[b],
                                  semg.at[b]).wait()

        def compute(b):
            hr, ar, dr = hrows.at[b], arows.at[b], drows.at[b]
            orr, er = orows.at[b], erows.at[b]

            @plsc.parallel_loop(0, CH, unroll=8)
            def edge_body(i):
                a = ar[i, :] + dr[i, :]
                e = jnp.exp(_leaky(a) - kv)
                er[i, :] = e
                for j in range(4):
                    orr[i, pl.ds(16 * j, 16)] = hr[i, pl.ds(16 * j, 16)] * e

        def issue_scatters(b):
            pltpu.async_copy(orows.at[b], acc_m.at[sdix.at[b]], sems.at[b],
                             add=True)
            pltpu.async_copy(erows.at[b], acc_e.at[sdix.at[b]], sems.at[b],
                             add=True)

        def wait_scatters(b):
            pltpu.make_async_copy(orows.at[b], acc_m.at[sdix.at[b]],
                                  sems.at[b]).wait()
            pltpu.make_async_copy(erows.at[b], acc_e.at[sdix.at[b]],
                                  sems.at[b]).wait()

        for b in range(NBUF - 1):
            issue_gathers(b, b)

        def ring_body(k4, carry):
            for b in range(NBUF):
                k = NBUF * k4 + b
                wait_gathers(b)

                @pl.when(k4 > 0)
                def _():
                    wait_scatters(b)

                pltpu.sync_copy(dst_hbm.at[pl.ds(wid * EPW + k * CH, CH)],
                                sdix.at[b])
                compute(b)
                issue_scatters(b)

                @pl.when(k + NBUF - 1 < NCHUNK)
                def _():
                    issue_gathers(k + NBUF - 1, (b + NBUF - 1) % NBUF)

            return carry

        lax.fori_loop(0, NCHUNK // NBUF, ring_body, 0)
        for b in range(NBUF):
            wait_scatters(b)
        plsc.subcore_barrier()
        pltpu.sync_copy(acc_m.at[pl.ds(s * STRIPE, STRIPE)],
                        m_out.at[c, pl.ds(s * STRIPE, STRIPE)])
        pltpu.sync_copy(acc_e.at[pl.ds(s * STRIPE, STRIPE)],
                        e_out.at[c, pl.ds(s * STRIPE, STRIPE)])

    return pl.kernel(
        body,
        out_type=[
            jax.ShapeDtypeStruct((NC, NPAD, HC), jnp.float32),
            jax.ShapeDtypeStruct((NC, NPAD, 16), jnp.float32),
        ],
        mesh=mesh,
        compiler_params=pltpu.CompilerParams(needs_layout_passes=False,
                                             use_tc_tiling_on_sc=False),
        scratch_types=[
            pltpu.VMEM((NBUF, CH), jnp.int32),
            pltpu.VMEM((NBUF, CH), jnp.int32),
            pltpu.VMEM((NBUF, CH), jnp.int32),
            pltpu.VMEM((NBUF, CH, HC), jnp.float32),
            pltpu.VMEM((NBUF, CH, 16), jnp.float32),
            pltpu.VMEM((NBUF, CH, 16), jnp.float32),
            pltpu.VMEM((NBUF, CH, HC), jnp.float32),
            pltpu.VMEM((NBUF, CH, 16), jnp.float32),
            pltpu.VMEM((16,), jnp.float32),
            pltpu.VMEM_SHARED((NPAD, HC), jnp.float32),
            pltpu.VMEM_SHARED((NPAD, 16), jnp.float32),
            pltpu.SemaphoreType.DMA((NBUF,)),
            pltpu.SemaphoreType.DMA((NBUF,)),
        ],
    )


import functools


@functools.lru_cache(maxsize=1)
def _get_edge_kernel():
    return _make_edge_kernel()


def _epi1_body(m_ref, e_ref, h_ref, as_ref, ad_ref, kv_ref, b_ref, r_ref,
               w2_ref, asm_ref, adm_ref, h2_ref, as2_ref, ad2_ref, mx_ref):
    i = pl.program_id(0)
    m = m_ref[...][0] + m_ref[...][1]
    e2 = e_ref[...][0] + e_ref[...][1]
    a = as_ref[...] + ad_ref[...]
    es = jnp.exp(_leaky(a) - kv_ref[...])
    den = jnp.dot(e2 + es, r_ref[...], preferred_element_type=jnp.float32)
    esb = jnp.dot(es, r_ref[...], preferred_element_type=jnp.float32)
    num = m + h_ref[...] * esb
    h1 = num / (den + 1e-16) + b_ref[...]
    h1e = jnp.where(h1 > 0, h1, jnp.exp(h1) - 1.0)
    h2 = jnp.dot(h1e, w2_ref[...], preferred_element_type=jnp.float32)
    a_s2 = jnp.dot(h2, asm_ref[...], preferred_element_type=jnp.float32)
    a_d2 = jnp.dot(h2, adm_ref[...], preferred_element_type=jnp.float32)
    h2_ref[...] = h2
    as2_ref[...] = a_s2
    ad2_ref[...] = a_d2

    @pl.when(i == 0)
    def _():
        mx_ref[...] = jnp.full((2, 16), -3.0e38, jnp.float32)

    upd = jnp.concatenate(
        [jnp.max(a_s2, axis=0, keepdims=True), jnp.max(a_d2, axis=0, keepdims=True)],
        axis=0,
    )
    mx_ref[...] = jnp.maximum(mx_ref[...], upd)


def _epi1(m1, e1, h1, as1, ad1, kv1, b1, r16, w2, asm2, adm2):
    return pl.pallas_call(
        _epi1_body,
        grid=(N // BM,),
        in_specs=[
            pl.BlockSpec((2, BM, HC), lambda i: (0, i, 0)),
            pl.BlockSpec((2, BM, 16), lambda i: (0, i, 0)),
            pl.BlockSpec((BM, HC), lambda i: (i, 0)),
            pl.BlockSpec((BM, 16), lambda i: (i, 0)),
            pl.BlockSpec((BM, 16), lambda i: (i, 0)),
            pl.BlockSpec((1, 16), lambda i: (0, 0)),
            pl.BlockSpec((1, HC), lambda i: (0, 0)),
            pl.BlockSpec((16, HC), lambda i: (0, 0)),
            pl.BlockSpec((HC, HC), lambda i: (0, 0)),
            pl.BlockSpec((HC, 16), lambda i: (0, 0)),
            pl.BlockSpec((HC, 16), lambda i: (0, 0)),
        ],
        out_specs=[
            pl.BlockSpec((BM, HC), lambda i: (i, 0)),
            pl.BlockSpec((BM, 16), lambda i: (i, 0)),
            pl.BlockSpec((BM, 16), lambda i: (i, 0)),
            pl.BlockSpec((2, 16), lambda i: (0, 0)),
        ],
        out_shape=[
            jax.ShapeDtypeStruct((N, HC), jnp.float32),
            jax.ShapeDtypeStruct((N, 16), jnp.float32),
            jax.ShapeDtypeStruct((N, 16), jnp.float32),
            jax.ShapeDtypeStruct((2, 16), jnp.float32),
        ],
    )(m1, e1, h1, as1, ad1, kv1, b1, r16, w2, asm2, adm2)


def _epi2_body(m_ref, e_ref, h_ref, as_ref, ad_ref, kv_ref, b_ref, r_ref,
               out_ref):
    m = m_ref[...][0] + m_ref[...][1]
    e2 = e_ref[...][0] + e_ref[...][1]
    a = as_ref[...] + ad_ref[...]
    es = jnp.exp(_leaky(a) - kv_ref[...])
    den = jnp.dot(e2 + es, r_ref[...], preferred_element_type=jnp.float32)
    esb = jnp.dot(es, r_ref[...], preferred_element_type=jnp.float32)
    num = m + h_ref[...] * esb
    o = num / (den + 1e-16) + b_ref[...]
    mx = jnp.max(o, axis=1, keepdims=True)
    z = o - mx
    lse = jnp.log(jnp.sum(jnp.exp(z), axis=1, keepdims=True))
    out_ref[...] = z - lse


def _epi2(m2, e2, h2, as2, ad2, kv2, b2, r16):
    return pl.pallas_call(
        _epi2_body,
        grid=(N // BM,),
        in_specs=[
            pl.BlockSpec((2, BM, HC), lambda i: (0, i, 0)),
            pl.BlockSpec((2, BM, 16), lambda i: (0, i, 0)),
            pl.BlockSpec((BM, HC), lambda i: (i, 0)),
            pl.BlockSpec((BM, 16), lambda i: (i, 0)),
            pl.BlockSpec((BM, 16), lambda i: (i, 0)),
            pl.BlockSpec((1, 16), lambda i: (0, 0)),
            pl.BlockSpec((1, HC), lambda i: (0, 0)),
            pl.BlockSpec((16, HC), lambda i: (0, 0)),
        ],
        out_specs=pl.BlockSpec((BM, HC), lambda i: (i, 0)),
        out_shape=jax.ShapeDtypeStruct((N, HC), jnp.float32),
    )(m2, e2, h2, as2, ad2, kv2, b2, r16)


_PERM = (jnp.arange(HC) % 8) * 8 + jnp.arange(HC) // 8


def _pack_mats1(att_src, att_dst):
    fs = att_src.reshape(HC)[_PERM]
    fd = att_dst.reshape(HC)[_PERM]
    rows = jnp.arange(HC)
    cols = jnp.arange(16)
    sel = ((cols[None, :] % 8) == (rows[:, None] % 8)).astype(jnp.float32)
    return fs[:, None] * sel, fd[:, None] * sel


def _pack_mats2(att_src, att_dst):
    fs = att_src.reshape(HC)
    fd = att_dst.reshape(HC)
    sel = jnp.ones((HC, 16), jnp.float32)
    return fs[:, None] * sel, fd[:, None] * sel


def _bcast_mat1():
    rows = jnp.arange(16)[:, None]
    t = jnp.arange(HC)[None, :]
    return (((t % 8) == rows) & (rows < 8)).astype(jnp.float32)


def _bcast_mat2():
    return (jnp.arange(16)[:, None] == 0).astype(jnp.float32) * jnp.ones(
        (1, HC), jnp.float32)


def kernel(x, edge_index, W1, att_src1, att_dst1, b1, W2, att_src2, att_dst2,
           b2):
    w1p = W1[:, _PERM]
    b1p = b1[_PERM]
    w2p = W2[_PERM, :]
    asm1, adm1 = _pack_mats1(att_src1, att_dst1)
    asm2, adm2 = _pack_mats2(att_src2, att_dst2)
    r16_1 = _bcast_mat1()
    r16_2 = _bcast_mat2()
    npad_e = EPAD - E
    srcp = jnp.concatenate([edge_index[0], jnp.zeros((npad_e,), jnp.int32)])
    dstp = jnp.concatenate([edge_index[1], jnp.full((npad_e,), N, jnp.int32)])
    z64 = jnp.zeros((NPAD, HC), jnp.float32)
    z16 = jnp.zeros((NPAD, 16), jnp.float32)
    b1r = b1p.reshape(1, HC)
    b2r = b2.reshape(1, HC)

    h1, as1, ad1, mx1 = _dense1(x, w1p, asm1, adm1)
    kv1 = _leaky(mx1[0] + mx1[1]).reshape(1, 16)
    m1, e1 = _get_edge_kernel()(h1, as1, ad1, srcp, dstp, kv1.reshape(16),
                                z64, z16)

    h2, as2, ad2, mx2 = _epi1(m1, e1, h1, as1, ad1, kv1, b1r, r16_1, w2p,
                              asm2, adm2)
    kv2 = _leaky(mx2[0] + mx2[1]).reshape(1, 16)
    m2, e2 = _get_edge_kernel()(h2, as2, ad2, srcp, dstp, kv2.reshape(16),
                                z64, z16)

    return _epi2(m2, e2, h2, as2, ad2, kv2, b2r, r16_2)


# speedup vs baseline: 1.1916x; 1.1916x over previous
import jax
import jax.numpy as jnp
from jax import lax
from jax.experimental import pallas as pl
from jax.experimental.pallas import tpu as pltpu
from jax.experimental.pallas import tpu_sc as plsc

N = 10000
F_IN = 256
HC = 64
NPAD = 10112
STRIPE = NPAD // 16
E = 160000
NC, NS = 2, 16
NW = NC * NS
EPW = 5120
EPAD = EPW * NW
CH = 128
NCHUNK = EPW // CH
NBUF = 2
BM = 400


def _leaky(x):
    return jnp.where(x >= 0, x, 0.2 * x)


def _dense1_body(x_ref, w_ref, asm_ref, adm_ref, h_ref, as_ref, ad_ref, mx_ref):
    i = pl.program_id(0)
    h = jnp.dot(x_ref[...], w_ref[...], preferred_element_type=jnp.float32)
    a_s = jnp.dot(h, asm_ref[...], preferred_element_type=jnp.float32)
    a_d = jnp.dot(h, adm_ref[...], preferred_element_type=jnp.float32)
    h_ref[...] = h
    as_ref[...] = a_s
    ad_ref[...] = a_d

    @pl.when(i == 0)
    def _():
        mx_ref[...] = jnp.full((2, 16), -3.0e38, jnp.float32)

    upd = jnp.concatenate(
        [jnp.max(a_s, axis=0, keepdims=True), jnp.max(a_d, axis=0, keepdims=True)],
        axis=0,
    )
    mx_ref[...] = jnp.maximum(mx_ref[...], upd)


def _dense1(x, w1, asm, adm):
    return pl.pallas_call(
        _dense1_body,
        grid=(N // BM,),
        in_specs=[
            pl.BlockSpec((BM, F_IN), lambda i: (i, 0)),
            pl.BlockSpec((F_IN, HC), lambda i: (0, 0)),
            pl.BlockSpec((HC, 16), lambda i: (0, 0)),
            pl.BlockSpec((HC, 16), lambda i: (0, 0)),
        ],
        out_specs=[
            pl.BlockSpec((BM, HC), lambda i: (i, 0)),
            pl.BlockSpec((BM, 16), lambda i: (i, 0)),
            pl.BlockSpec((BM, 16), lambda i: (i, 0)),
            pl.BlockSpec((2, 16), lambda i: (0, 0)),
        ],
        out_shape=[
            jax.ShapeDtypeStruct((N, HC), jnp.float32),
            jax.ShapeDtypeStruct((N, 16), jnp.float32),
            jax.ShapeDtypeStruct((N, 16), jnp.float32),
            jax.ShapeDtypeStruct((2, 16), jnp.float32),
        ],
    )(x, w1, asm, adm)


def _make_edge_kernel():
    mesh = plsc.VectorSubcoreMesh(core_axis_name="c", subcore_axis_name="s",
                                  num_cores=NC, num_subcores=NS)

    def body(h_hbm, as_hbm, ad_hbm, src_hbm, dst_hbm, kv_hbm, z64_hbm, z16_hbm,
             m_out, e_out, sidx, didx, sdix, hrows, arows, drows, orows,
             erows, kv_v, acc_m, acc_e, semg, sems):
        c = lax.axis_index("c")
        s = lax.axis_index("s")
        wid = s * NC + c
        pltpu.sync_copy(z64_hbm.at[pl.ds(s * STRIPE, STRIPE)],
                        acc_m.at[pl.ds(s * STRIPE, STRIPE)])
        pltpu.sync_copy(z16_hbm.at[pl.ds(s * STRIPE, STRIPE)],
                        acc_e.at[pl.ds(s * STRIPE, STRIPE)])
        pltpu.sync_copy(kv_hbm, kv_v)
        plsc.subcore_barrier()

        kv = kv_v[...]

        def issue_gathers(k, b):
            base = wid * EPW + k * CH
            pltpu.sync_copy(src_hbm.at[pl.ds(base, CH)], sidx.at[b])
            pltpu.sync_copy(dst_hbm.at[pl.ds(base, CH)], didx.at[b])
            pltpu.async_copy(h_hbm.at[sidx.at[b]], hrows.at[b], semg.at[b])
            pltpu.async_copy(as_hbm.at[sidx.at[b]], arows.at[b], semg.at[b])
            pltpu.async_copy(ad_hbm.at[didx.at[b]], drows.at[b], semg.at[b])

        def wait_gathers(b):
            pltpu.make_async_copy(h_hbm.at[sidx.at[b]], hrows.at[b],
                                  semg.at[b]).wait()
            pltpu.make_async_copy(as_hbm.at[sidx.at[b]], arows.at[b],
                                  semg.at[b]).wait()
            pltpu.make_async_copy(ad_hbm.at[didx.at[b]], drows.at[b],
                                  semg.at[b]).wait()

        def compute(b):
            hr, ar, dr = hrows.at[b], arows.at[b], drows.at[b]
            orr, er = orows.at[b], erows.at[b]

            @plsc.parallel_loop(0, CH, unroll=8)
            def edge_body(i):
                a = ar[i, :] + dr[i, :]
                e = jnp.exp(_leaky(a) - kv)
                er[i, :] = e
                for j in range(4):
                    orr[i, pl.ds(16 * j, 16)] = hr[i, pl.ds(16 * j, 16)] * e

        def issue_scatters(b):
            pltpu.async_copy(orows.at[b], acc_m.at[sdix.at[b]], sems.at[b],
                             add=True)
            pltpu.async_copy(erows.at[b], acc_e.at[sdix.at[b]], sems.at[b],
                             add=True)

        def wait_scatters(b):
            pltpu.make_async_copy(orows.at[b], acc_m.at[sdix.at[b]],
                                  sems.at[b]).wait()
            pltpu.make_async_copy(erows.at[b], acc_e.at[sdix.at[b]],
                                  sems.at[b]).wait()

        for b in range(NBUF - 1):
            issue_gathers(b, b)

        def ring_body(k4, carry):
            for b in range(NBUF):
                k = NBUF * k4 + b
                wait_gathers(b)

                @pl.when(k4 > 0)
                def _():
                    wait_scatters(b)

                pltpu.sync_copy(dst_hbm.at[pl.ds(wid * EPW + k * CH, CH)],
                                sdix.at[b])
                compute(b)
                issue_scatters(b)

                @pl.when(k + NBUF - 1 < NCHUNK)
                def _():
                    issue_gathers(k + NBUF - 1, (b + NBUF - 1) % NBUF)

            return carry

        lax.fori_loop(0, NCHUNK // NBUF, ring_body, 0)
        for b in range(NBUF):
            wait_scatters(b)
        plsc.subcore_barrier()
        pltpu.sync_copy(acc_m.at[pl.ds(s * STRIPE, STRIPE)],
                        m_out.at[c, pl.ds(s * STRIPE, STRIPE)])
        pltpu.sync_copy(acc_e.at[pl.ds(s * STRIPE, STRIPE)],
                        e_out.at[c, pl.ds(s * STRIPE, STRIPE)])

    return pl.kernel(
        body,
        out_type=[
            jax.ShapeDtypeStruct((NC, NPAD, HC), jnp.float32),
            jax.ShapeDtypeStruct((NC, NPAD, 16), jnp.float32),
        ],
        mesh=mesh,
        compiler_params=pltpu.CompilerParams(needs_layout_passes=False,
                                             use_tc_tiling_on_sc=False),
        scratch_types=[
            pltpu.VMEM((NBUF, CH), jnp.int32),
            pltpu.VMEM((NBUF, CH), jnp.int32),
            pltpu.VMEM((NBUF, CH), jnp.int32),
            pltpu.VMEM((NBUF, CH, HC), jnp.float32),
            pltpu.VMEM((NBUF, CH, 16), jnp.float32),
            pltpu.VMEM((NBUF, CH, 16), jnp.float32),
            pltpu.VMEM((NBUF, CH, HC), jnp.float32),
            pltpu.VMEM((NBUF, CH, 16), jnp.float32),
            pltpu.VMEM((16,), jnp.float32),
            pltpu.VMEM_SHARED((NPAD, HC), jnp.float32),
            pltpu.VMEM_SHARED((NPAD, 16), jnp.float32),
            pltpu.SemaphoreType.DMA((NBUF,)),
            pltpu.SemaphoreType.DMA((NBUF,)),
        ],
    )


import functools


@functools.lru_cache(maxsize=1)
def _get_edge_kernel():
    return _make_edge_kernel()


def _epi1_body(m_ref, e_ref, h_ref, as_ref, ad_ref, kv_ref, b_ref, r_ref,
               w2_ref, asm_ref, adm_ref, h2_ref, as2_ref, ad2_ref, mx_ref):
    i = pl.program_id(0)
    m = m_ref[...][0] + m_ref[...][1]
    e2 = e_ref[...][0] + e_ref[...][1]
    a = as_ref[...] + ad_ref[...]
    es = jnp.exp(_leaky(a) - kv_ref[...])
    den = jnp.dot(e2 + es, r_ref[...], preferred_element_type=jnp.float32)
    esb = jnp.dot(es, r_ref[...], preferred_element_type=jnp.float32)
    num = m + h_ref[...] * esb
    h1 = num / (den + 1e-16) + b_ref[...]
    h1e = jnp.where(h1 > 0, h1, jnp.exp(h1) - 1.0)
    h2 = jnp.dot(h1e, w2_ref[...], preferred_element_type=jnp.float32)
    a_s2 = jnp.dot(h2, asm_ref[...], preferred_element_type=jnp.float32)
    a_d2 = jnp.dot(h2, adm_ref[...], preferred_element_type=jnp.float32)
    h2_ref[...] = h2
    as2_ref[...] = a_s2
    ad2_ref[...] = a_d2

    @pl.when(i == 0)
    def _():
        mx_ref[...] = jnp.full((2, 16), -3.0e38, jnp.float32)

    upd = jnp.concatenate(
        [jnp.max(a_s2, axis=0, keepdims=True), jnp.max(a_d2, axis=0, keepdims=True)],
        axis=0,
    )
    mx_ref[...] = jnp.maximum(mx_ref[...], upd)


def _epi1(m1, e1, h1, as1, ad1, kv1, b1, r16, w2, asm2, adm2):
    return pl.pallas_call(
        _epi1_body,
        grid=(N // BM,),
        in_specs=[
            pl.BlockSpec((2, BM, HC), lambda i: (0, i, 0)),
            pl.BlockSpec((2, BM, 16), lambda i: (0, i, 0)),
            pl.BlockSpec((BM, HC), lambda i: (i, 0)),
            pl.BlockSpec((BM, 16), lambda i: (i, 0)),
            pl.BlockSpec((BM, 16), lambda i: (i, 0)),
            pl.BlockSpec((1, 16), lambda i: (0, 0)),
            pl.BlockSpec((1, HC), lambda i: (0, 0)),
            pl.BlockSpec((16, HC), lambda i: (0, 0)),
            pl.BlockSpec((HC, HC), lambda i: (0, 0)),
            pl.BlockSpec((HC, 16), lambda i: (0, 0)),
            pl.BlockSpec((HC, 16), lambda i: (0, 0)),
        ],
        out_specs=[
            pl.BlockSpec((BM, HC), lambda i: (i, 0)),
            pl.BlockSpec((BM, 16), lambda i: (i, 0)),
            pl.BlockSpec((BM, 16), lambda i: (i, 0)),
            pl.BlockSpec((2, 16), lambda i: (0, 0)),
        ],
        out_shape=[
            jax.ShapeDtypeStruct((N, HC), jnp.float32),
            jax.ShapeDtypeStruct((N, 16), jnp.float32),
            jax.ShapeDtypeStruct((N, 16), jnp.float32),
            jax.ShapeDtypeStruct((2, 16), jnp.float32),
        ],
    )(m1, e1, h1, as1, ad1, kv1, b1, r16, w2, asm2, adm2)


def _epi2_body(m_ref, e_ref, h_ref, as_ref, ad_ref, kv_ref, b_ref, r_ref,
               out_ref):
    m = m_ref[...][0] + m_ref[...][1]
    e2 = e_ref[...][0] + e_ref[...][1]
    a = as_ref[...] + ad_ref[...]
    es = jnp.exp(_leaky(a) - kv_ref[...])
    den = jnp.dot(e2 + es, r_ref[...], preferred_element_type=jnp.float32)
    esb = jnp.dot(es, r_ref[...], preferred_element_type=jnp.float32)
    num = m + h_ref[...] * esb
    o = num / (den + 1e-16) + b_ref[...]
    mx = jnp.max(o, axis=1, keepdims=True)
    z = o - mx
    lse = jnp.log(jnp.sum(jnp.exp(z), axis=1, keepdims=True))
    out_ref[...] = z - lse


def _epi2(m2, e2, h2, as2, ad2, kv2, b2, r16):
    return pl.pallas_call(
        _epi2_body,
        grid=(N // BM,),
        in_specs=[
            pl.BlockSpec((2, BM, HC), lambda i: (0, i, 0)),
            pl.BlockSpec((2, BM, 16), lambda i: (0, i, 0)),
            pl.BlockSpec((BM, HC), lambda i: (i, 0)),
            pl.BlockSpec((BM, 16), lambda i: (i, 0)),
            pl.BlockSpec((BM, 16), lambda i: (i, 0)),
            pl.BlockSpec((1, 16), lambda i: (0, 0)),
            pl.BlockSpec((1, HC), lambda i: (0, 0)),
            pl.BlockSpec((16, HC), lambda i: (0, 0)),
        ],
        out_specs=pl.BlockSpec((BM, HC), lambda i: (i, 0)),
        out_shape=jax.ShapeDtypeStruct((N, HC), jnp.float32),
    )(m2, e2, h2, as2, ad2, kv2, b2, r16)


_PERM = (jnp.arange(HC) % 8) * 8 + jnp.arange(HC) // 8


def _pack_mats1(att_src, att_dst):
    fs = att_src.reshape(HC)[_PERM]
    fd = att_dst.reshape(HC)[_PERM]
    rows = jnp.arange(HC)
    cols = jnp.arange(16)
    sel = ((cols[None, :] % 8) == (rows[:, None] % 8)).astype(jnp.float32)
    return fs[:, None] * sel, fd[:, None] * sel


def _pack_mats2(att_src, att_dst):
    fs = att_src.reshape(HC)
    fd = att_dst.reshape(HC)
    sel = jnp.ones((HC, 16), jnp.float32)
    return fs[:, None] * sel, fd[:, None] * sel


def _bcast_mat1():
    rows = jnp.arange(16)[:, None]
    t = jnp.arange(HC)[None, :]
    return (((t % 8) == rows) & (rows < 8)).astype(jnp.float32)


def _bcast_mat2():
    return (jnp.arange(16)[:, None] == 0).astype(jnp.float32) * jnp.ones(
        (1, HC), jnp.float32)


def kernel(x, edge_index, W1, att_src1, att_dst1, b1, W2, att_src2, att_dst2,
           b2):
    w1p = W1[:, _PERM]
    b1p = b1[_PERM]
    w2p = W2[_PERM, :]
    asm1, adm1 = _pack_mats1(att_src1, att_dst1)
    asm2, adm2 = _pack_mats2(att_src2, att_dst2)
    r16_1 = _bcast_mat1()
    r16_2 = _bcast_mat2()
    npad_e = EPAD - E
    srcp = jnp.concatenate([edge_index[0], jnp.zeros((npad_e,), jnp.int32)])
    dstp = jnp.concatenate([edge_index[1], jnp.full((npad_e,), N, jnp.int32)])
    z64 = jnp.zeros((NPAD, HC), jnp.float32)
    z16 = jnp.zeros((NPAD, 16), jnp.float32)
    b1r = b1p.reshape(1, HC)
    b2r = b2.reshape(1, HC)

    h1, as1, ad1, mx1 = _dense1(x, w1p, asm1, adm1)
    kv1 = _leaky(mx1[0] + mx1[1]).reshape(1, 16)
    m1, e1 = _get_edge_kernel()(h1, as1, ad1, srcp, dstp, kv1.reshape(16),
                                z64, z16)

    h2, as2, ad2, mx2 = _epi1(m1, e1, h1, as1, ad1, kv1, b1r, r16_1, w2p,
                              asm2, adm2)
    kv2 = _leaky(mx2[0] + mx2[1]).reshape(1, 16)
    m2, e2 = _get_edge_kernel()(h2, as2, ad2, srcp, dstp, kv2.reshape(16),
                                z64, z16)

    return _epi2(m2, e2, h2, as2, ad2, kv2, b2r, r16_2)


# speedup vs baseline: 1.5945x; 1.3381x over previous
import jax
import jax.numpy as jnp
from jax import lax
from jax.experimental import pallas as pl
from jax.experimental.pallas import tpu as pltpu
from jax.experimental.pallas import tpu_sc as plsc

N = 10000
F_IN = 256
HC = 64
NPAD = 10112
STRIPE = NPAD // 16
E = 160000
NC, NS = 2, 16
NW = NC * NS
EPW = 5120
EPAD = EPW * NW
CH = 128
NCHUNK = EPW // CH
NBUF = 2
BM = 400


def _leaky(x):
    return jnp.where(x >= 0, x, 0.2 * x)


def _dense1_body(x_ref, w_ref, asm_ref, adm_ref, h_ref, as_ref, ad_ref, mx_ref):
    i = pl.program_id(0)
    h = jnp.dot(x_ref[...], w_ref[...], preferred_element_type=jnp.float32)
    a_s = jnp.dot(h, asm_ref[...], preferred_element_type=jnp.float32)
    a_d = jnp.dot(h, adm_ref[...], preferred_element_type=jnp.float32)
    h_ref[...] = h
    as_ref[...] = a_s
    ad_ref[...] = a_d

    @pl.when(i == 0)
    def _():
        mx_ref[...] = jnp.full((2, 16), -3.0e38, jnp.float32)

    upd = jnp.concatenate(
        [jnp.max(a_s, axis=0, keepdims=True), jnp.max(a_d, axis=0, keepdims=True)],
        axis=0,
    )
    mx_ref[...] = jnp.maximum(mx_ref[...], upd)


def _dense1(x, w1, asm, adm):
    return pl.pallas_call(
        _dense1_body,
        grid=(N // BM,),
        in_specs=[
            pl.BlockSpec((BM, F_IN), lambda i: (i, 0)),
            pl.BlockSpec((F_IN, HC), lambda i: (0, 0)),
            pl.BlockSpec((HC, 16), lambda i: (0, 0)),
            pl.BlockSpec((HC, 16), lambda i: (0, 0)),
        ],
        out_specs=[
            pl.BlockSpec((BM, HC), lambda i: (i, 0)),
            pl.BlockSpec((BM, 16), lambda i: (i, 0)),
            pl.BlockSpec((BM, 16), lambda i: (i, 0)),
            pl.BlockSpec((2, 16), lambda i: (0, 0)),
        ],
        out_shape=[
            jax.ShapeDtypeStruct((N, HC), jnp.float32),
            jax.ShapeDtypeStruct((N, 16), jnp.float32),
            jax.ShapeDtypeStruct((N, 16), jnp.float32),
            jax.ShapeDtypeStruct((2, 16), jnp.float32),
        ],
    )(x, w1, asm, adm)


def _make_edge_kernel():
    mesh = plsc.VectorSubcoreMesh(core_axis_name="c", subcore_axis_name="s",
                                  num_cores=NC, num_subcores=NS)

    def body(h_hbm, as_hbm, ad_hbm, src_hbm, dst_hbm, kv_hbm, z64_hbm, z16_hbm,
             m_out, e_out, sidx, didx, sdix, hrows, arows, drows, orows,
             erows, kv_v, acc_m, acc_e, semg, sems):
        c = lax.axis_index("c")
        s = lax.axis_index("s")
        wid = s * NC + c
        pltpu.sync_copy(z64_hbm.at[pl.ds(s * STRIPE, STRIPE)],
                        acc_m.at[pl.ds(s * STRIPE, STRIPE)])
        pltpu.sync_copy(z16_hbm.at[pl.ds(s * STRIPE, STRIPE)],
                        acc_e.at[pl.ds(s * STRIPE, STRIPE)])
        pltpu.sync_copy(kv_hbm, kv_v)
        plsc.subcore_barrier()

        kv = kv_v[...]

        def issue_gathers(k, b):
            base = wid * EPW + k * CH
            pltpu.sync_copy(src_hbm.at[pl.ds(base, CH)], sidx.at[b])
            pltpu.sync_copy(dst_hbm.at[pl.ds(base, CH)], didx.at[b])
            pltpu.async_copy(h_hbm.at[sidx.at[b]], hrows.at[b], semg.at[b])
            pltpu.async_copy(as_hbm.at[sidx.at[b]], arows.at[b], semg.at[b])
            pltpu.async_copy(ad_hbm.at[didx.at[b]], drows.at[b], semg.at[b])

        def wait_gathers(b):
            pltpu.make_async_copy(h_hbm.at[sidx.at[b]], hrows.at[b],
                                  semg.at[b]).wait()
            pltpu.make_async_copy(as_hbm.at[sidx.at[b]], arows.at[b],
                                  semg.at[b]).wait()
            pltpu.make_async_copy(ad_hbm.at[didx.at[b]], drows.at[b],
                                  semg.at[b]).wait()

        def compute(b):
            hr, ar, dr = hrows.at[b], arows.at[b], drows.at[b]
            orr, er = orows.at[b], erows.at[b]

            @plsc.parallel_loop(0, CH, unroll=8)
            def edge_body(i):
                a = ar[i, :] + dr[i, :]
                e = jnp.exp(_leaky(a) - kv)
                er[i, :] = e
                for j in range(4):
                    orr[i, pl.ds(16 * j, 16)] = hr[i, pl.ds(16 * j, 16)] * e

        def issue_scatters(b):
            pltpu.async_copy(orows.at[b], acc_m.at[sdix.at[b]], sems.at[b],
                             add=True)
            pltpu.async_copy(erows.at[b], acc_e.at[sdix.at[b]], sems.at[b],
                             add=True)

        def wait_scatters(b):
            pltpu.make_async_copy(orows.at[b], acc_m.at[sdix.at[b]],
                                  sems.at[b]).wait()
            pltpu.make_async_copy(erows.at[b], acc_e.at[sdix.at[b]],
                                  sems.at[b]).wait()

        for b in range(NBUF - 1):
            issue_gathers(b, b)

        def ring_body(k4, carry):
            for b in range(NBUF):
                k = NBUF * k4 + b

                @pl.when(k + NBUF - 1 < NCHUNK)
                def _():
                    issue_gathers(k + NBUF - 1, (b + NBUF - 1) % NBUF)

                wait_gathers(b)

                @pl.when(k4 > 0)
                def _():
                    wait_scatters(b)

                pltpu.sync_copy(dst_hbm.at[pl.ds(wid * EPW + k * CH, CH)],
                                sdix.at[b])
                compute(b)
                issue_scatters(b)

            return carry

        lax.fori_loop(0, NCHUNK // NBUF, ring_body, 0)
        for b in range(NBUF):
            wait_scatters(b)
        plsc.subcore_barrier()
        pltpu.sync_copy(acc_m.at[pl.ds(s * STRIPE, STRIPE)],
                        m_out.at[c, pl.ds(s * STRIPE, STRIPE)])
        pltpu.sync_copy(acc_e.at[pl.ds(s * STRIPE, STRIPE)],
                        e_out.at[c, pl.ds(s * STRIPE, STRIPE)])

    return pl.kernel(
        body,
        out_type=[
            jax.ShapeDtypeStruct((NC, NPAD, HC), jnp.float32),
            jax.ShapeDtypeStruct((NC, NPAD, 16), jnp.float32),
        ],
        mesh=mesh,
        compiler_params=pltpu.CompilerParams(needs_layout_passes=False,
                                             use_tc_tiling_on_sc=False),
        scratch_types=[
            pltpu.VMEM((NBUF, CH), jnp.int32),
            pltpu.VMEM((NBUF, CH), jnp.int32),
            pltpu.VMEM((NBUF, CH), jnp.int32),
            pltpu.VMEM((NBUF, CH, HC), jnp.float32),
            pltpu.VMEM((NBUF, CH, 16), jnp.float32),
            pltpu.VMEM((NBUF, CH, 16), jnp.float32),
            pltpu.VMEM((NBUF, CH, HC), jnp.float32),
            pltpu.VMEM((NBUF, CH, 16), jnp.float32),
            pltpu.VMEM((16,), jnp.float32),
            pltpu.VMEM_SHARED((NPAD, HC), jnp.float32),
            pltpu.VMEM_SHARED((NPAD, 16), jnp.float32),
            pltpu.SemaphoreType.DMA((NBUF,)),
            pltpu.SemaphoreType.DMA((NBUF,)),
        ],
    )


import functools


@functools.lru_cache(maxsize=1)
def _get_edge_kernel():
    return _make_edge_kernel()


def _epi1_body(m_ref, e_ref, h_ref, as_ref, ad_ref, kv_ref, b_ref, r_ref,
               w2_ref, asm_ref, adm_ref, h2_ref, as2_ref, ad2_ref, mx_ref):
    i = pl.program_id(0)
    m = m_ref[...][0] + m_ref[...][1]
    e2 = e_ref[...][0] + e_ref[...][1]
    a = as_ref[...] + ad_ref[...]
    es = jnp.exp(_leaky(a) - kv_ref[...])
    den = jnp.dot(e2 + es, r_ref[...], preferred_element_type=jnp.float32)
    esb = jnp.dot(es, r_ref[...], preferred_element_type=jnp.float32)
    num = m + h_ref[...] * esb
    h1 = num / (den + 1e-16) + b_ref[...]
    h1e = jnp.where(h1 > 0, h1, jnp.exp(h1) - 1.0)
    h2 = jnp.dot(h1e, w2_ref[...], preferred_element_type=jnp.float32)
    a_s2 = jnp.dot(h2, asm_ref[...], preferred_element_type=jnp.float32)
    a_d2 = jnp.dot(h2, adm_ref[...], preferred_element_type=jnp.float32)
    h2_ref[...] = h2
    as2_ref[...] = a_s2
    ad2_ref[...] = a_d2

    @pl.when(i == 0)
    def _():
        mx_ref[...] = jnp.full((2, 16), -3.0e38, jnp.float32)

    upd = jnp.concatenate(
        [jnp.max(a_s2, axis=0, keepdims=True), jnp.max(a_d2, axis=0, keepdims=True)],
        axis=0,
    )
    mx_ref[...] = jnp.maximum(mx_ref[...], upd)


def _epi1(m1, e1, h1, as1, ad1, kv1, b1, r16, w2, asm2, adm2):
    return pl.pallas_call(
        _epi1_body,
        grid=(N // BM,),
        in_specs=[
            pl.BlockSpec((2, BM, HC), lambda i: (0, i, 0)),
            pl.BlockSpec((2, BM, 16), lambda i: (0, i, 0)),
            pl.BlockSpec((BM, HC), lambda i: (i, 0)),
            pl.BlockSpec((BM, 16), lambda i: (i, 0)),
            pl.BlockSpec((BM, 16), lambda i: (i, 0)),
            pl.BlockSpec((1, 16), lambda i: (0, 0)),
            pl.BlockSpec((1, HC), lambda i: (0, 0)),
            pl.BlockSpec((16, HC), lambda i: (0, 0)),
            pl.BlockSpec((HC, HC), lambda i: (0, 0)),
            pl.BlockSpec((HC, 16), lambda i: (0, 0)),
            pl.BlockSpec((HC, 16), lambda i: (0, 0)),
        ],
        out_specs=[
            pl.BlockSpec((BM, HC), lambda i: (i, 0)),
            pl.BlockSpec((BM, 16), lambda i: (i, 0)),
            pl.BlockSpec((BM, 16), lambda i: (i, 0)),
            pl.BlockSpec((2, 16), lambda i: (0, 0)),
        ],
        out_shape=[
            jax.ShapeDtypeStruct((N, HC), jnp.float32),
            jax.ShapeDtypeStruct((N, 16), jnp.float32),
            jax.ShapeDtypeStruct((N, 16), jnp.float32),
            jax.ShapeDtypeStruct((2, 16), jnp.float32),
        ],
    )(m1, e1, h1, as1, ad1, kv1, b1, r16, w2, asm2, adm2)


def _epi2_body(m_ref, e_ref, h_ref, as_ref, ad_ref, kv_ref, b_ref, r_ref,
               out_ref):
    m = m_ref[...][0] + m_ref[...][1]
    e2 = e_ref[...][0] + e_ref[...][1]
    a = as_ref[...] + ad_ref[...]
    es = jnp.exp(_leaky(a) - kv_ref[...])
    den = jnp.dot(e2 + es, r_ref[...], preferred_element_type=jnp.float32)
    esb = jnp.dot(es, r_ref[...], preferred_element_type=jnp.float32)
    num = m + h_ref[...] * esb
    o = num / (den + 1e-16) + b_ref[...]
    mx = jnp.max(o, axis=1, keepdims=True)
    z = o - mx
    lse = jnp.log(jnp.sum(jnp.exp(z), axis=1, keepdims=True))
    out_ref[...] = z - lse


def _epi2(m2, e2, h2, as2, ad2, kv2, b2, r16):
    return pl.pallas_call(
        _epi2_body,
        grid=(N // BM,),
        in_specs=[
            pl.BlockSpec((2, BM, HC), lambda i: (0, i, 0)),
            pl.BlockSpec((2, BM, 16), lambda i: (0, i, 0)),
            pl.BlockSpec((BM, HC), lambda i: (i, 0)),
            pl.BlockSpec((BM, 16), lambda i: (i, 0)),
            pl.BlockSpec((BM, 16), lambda i: (i, 0)),
            pl.BlockSpec((1, 16), lambda i: (0, 0)),
            pl.BlockSpec((1, HC), lambda i: (0, 0)),
            pl.BlockSpec((16, HC), lambda i: (0, 0)),
        ],
        out_specs=pl.BlockSpec((BM, HC), lambda i: (i, 0)),
        out_shape=jax.ShapeDtypeStruct((N, HC), jnp.float32),
    )(m2, e2, h2, as2, ad2, kv2, b2, r16)


_PERM = (jnp.arange(HC) % 8) * 8 + jnp.arange(HC) // 8


def _pack_mats1(att_src, att_dst):
    fs = att_src.reshape(HC)[_PERM]
    fd = att_dst.reshape(HC)[_PERM]
    rows = jnp.arange(HC)
    cols = jnp.arange(16)
    sel = ((cols[None, :] % 8) == (rows[:, None] % 8)).astype(jnp.float32)
    return fs[:, None] * sel, fd[:, None] * sel


def _pack_mats2(att_src, att_dst):
    fs = att_src.reshape(HC)
    fd = att_dst.reshape(HC)
    sel = jnp.ones((HC, 16), jnp.float32)
    return fs[:, None] * sel, fd[:, None] * sel


def _bcast_mat1():
    rows = jnp.arange(16)[:, None]
    t = jnp.arange(HC)[None, :]
    return (((t % 8) == rows) & (rows < 8)).astype(jnp.float32)


def _bcast_mat2():
    return (jnp.arange(16)[:, None] == 0).astype(jnp.float32) * jnp.ones(
        (1, HC), jnp.float32)


def kernel(x, edge_index, W1, att_src1, att_dst1, b1, W2, att_src2, att_dst2,
           b2):
    w1p = W1[:, _PERM]
    b1p = b1[_PERM]
    w2p = W2[_PERM, :]
    asm1, adm1 = _pack_mats1(att_src1, att_dst1)
    asm2, adm2 = _pack_mats2(att_src2, att_dst2)
    r16_1 = _bcast_mat1()
    r16_2 = _bcast_mat2()
    npad_e = EPAD - E
    srcp = jnp.concatenate([edge_index[0], jnp.zeros((npad_e,), jnp.int32)])
    dstp = jnp.concatenate([edge_index[1], jnp.full((npad_e,), N, jnp.int32)])
    z64 = jnp.zeros((NPAD, HC), jnp.float32)
    z16 = jnp.zeros((NPAD, 16), jnp.float32)
    b1r = b1p.reshape(1, HC)
    b2r = b2.reshape(1, HC)

    h1, as1, ad1, mx1 = _dense1(x, w1p, asm1, adm1)
    kv1 = _leaky(mx1[0] + mx1[1]).reshape(1, 16)
    m1, e1 = _get_edge_kernel()(h1, as1, ad1, srcp, dstp, kv1.reshape(16),
                                z64, z16)

    h2, as2, ad2, mx2 = _epi1(m1, e1, h1, as1, ad1, kv1, b1r, r16_1, w2p,
                              asm2, adm2)
    kv2 = _leaky(mx2[0] + mx2[1]).reshape(1, 16)
    m2, e2 = _get_edge_kernel()(h2, as2, ad2, srcp, dstp, kv2.reshape(16),
                                z64, z16)

    return _epi2(m2, e2, h2, as2, ad2, kv2, b2r, r16_2)


# speedup vs baseline: 1.6981x; 1.0649x over previous
import jax
import jax.numpy as jnp
from jax import lax
from jax.experimental import pallas as pl
from jax.experimental.pallas import tpu as pltpu
from jax.experimental.pallas import tpu_sc as plsc

N = 10000
F_IN = 256
HC = 64
NPAD = 10112
STRIPE = NPAD // 16
E = 160000
NC, NS = 2, 16
NW = NC * NS
EPW = 5120
EPAD = EPW * NW
EPW0 = 6144
EPW1 = 2 * EPW - EPW0
CH = 128
NCHUNK = EPW // CH
NBUF = 2
BM = 400


def _leaky(x):
    return jnp.where(x >= 0, x, 0.2 * x)


def _dense1_body(x_ref, w_ref, asm_ref, adm_ref, h_ref, as_ref, ad_ref, mx_ref):
    i = pl.program_id(0)
    h = jnp.dot(x_ref[...], w_ref[...], preferred_element_type=jnp.float32)
    a_s = jnp.dot(h, asm_ref[...], preferred_element_type=jnp.float32)
    a_d = jnp.dot(h, adm_ref[...], preferred_element_type=jnp.float32)
    h_ref[...] = h
    as_ref[...] = a_s
    ad_ref[...] = a_d

    @pl.when(i == 0)
    def _():
        mx_ref[...] = jnp.full((2, 16), -3.0e38, jnp.float32)

    upd = jnp.concatenate(
        [jnp.max(a_s, axis=0, keepdims=True), jnp.max(a_d, axis=0, keepdims=True)],
        axis=0,
    )
    mx_ref[...] = jnp.maximum(mx_ref[...], upd)


def _dense1(x, w1, asm, adm):
    return pl.pallas_call(
        _dense1_body,
        grid=(N // BM,),
        in_specs=[
            pl.BlockSpec((BM, F_IN), lambda i: (i, 0)),
            pl.BlockSpec((F_IN, HC), lambda i: (0, 0)),
            pl.BlockSpec((HC, 16), lambda i: (0, 0)),
            pl.BlockSpec((HC, 16), lambda i: (0, 0)),
        ],
        out_specs=[
            pl.BlockSpec((BM, HC), lambda i: (i, 0)),
            pl.BlockSpec((BM, 16), lambda i: (i, 0)),
            pl.BlockSpec((BM, 16), lambda i: (i, 0)),
            pl.BlockSpec((2, 16), lambda i: (0, 0)),
        ],
        out_shape=[
            jax.ShapeDtypeStruct((N, HC), jnp.float32),
            jax.ShapeDtypeStruct((N, 16), jnp.float32),
            jax.ShapeDtypeStruct((N, 16), jnp.float32),
            jax.ShapeDtypeStruct((2, 16), jnp.float32),
        ],
    )(x, w1, asm, adm)


def _make_edge_kernel():
    mesh = plsc.VectorSubcoreMesh(core_axis_name="c", subcore_axis_name="s",
                                  num_cores=NC, num_subcores=NS)

    def body(h_hbm, as_hbm, ad_hbm, src_hbm, dst_hbm, kv_hbm, z64_hbm, z16_hbm,
             m_out, e_out, sidx, didx, sdix, hrows, arows, drows, orows,
             erows, kv_v, acc_m, acc_e, semg, sems):
        c = lax.axis_index("c")
        s = lax.axis_index("s")
        ew = jnp.where(c == 0, EPW0, EPW1)
        woff = jnp.where(c == 0, 0, NS * EPW0) + s * ew
        nch = ew // CH
        pltpu.sync_copy(z64_hbm.at[pl.ds(s * STRIPE, STRIPE)],
                        acc_m.at[pl.ds(s * STRIPE, STRIPE)])
        pltpu.sync_copy(z16_hbm.at[pl.ds(s * STRIPE, STRIPE)],
                        acc_e.at[pl.ds(s * STRIPE, STRIPE)])
        pltpu.sync_copy(kv_hbm, kv_v)
        plsc.subcore_barrier()

        kv = kv_v[...]

        def issue_gathers(k, b):
            base = woff + k * CH
            pltpu.sync_copy(src_hbm.at[pl.ds(base, CH)], sidx.at[b])
            pltpu.sync_copy(dst_hbm.at[pl.ds(base, CH)], didx.at[b])
            pltpu.async_copy(h_hbm.at[sidx.at[b]], hrows.at[b], semg.at[b])
            pltpu.async_copy(as_hbm.at[sidx.at[b]], arows.at[b], semg.at[b])
            pltpu.async_copy(ad_hbm.at[didx.at[b]], drows.at[b], semg.at[b])

        def wait_gathers(b):
            pltpu.make_async_copy(h_hbm.at[sidx.at[b]], hrows.at[b],
                                  semg.at[b]).wait()
            pltpu.make_async_copy(as_hbm.at[sidx.at[b]], arows.at[b],
                                  semg.at[b]).wait()
            pltpu.make_async_copy(ad_hbm.at[didx.at[b]], drows.at[b],
                                  semg.at[b]).wait()

        def compute(b):
            hr, ar, dr = hrows.at[b], arows.at[b], drows.at[b]
            orr, er = orows.at[b], erows.at[b]

            @plsc.parallel_loop(0, CH, unroll=8)
            def edge_body(i):
                a = ar[i, :] + dr[i, :]
                e = jnp.exp(_leaky(a) - kv)
                er[i, :] = e
                for j in range(4):
                    orr[i, pl.ds(16 * j, 16)] = hr[i, pl.ds(16 * j, 16)] * e

        def issue_scatters(b):
            pltpu.async_copy(orows.at[b], acc_m.at[sdix.at[b]], sems.at[b],
                             add=True)
            pltpu.async_copy(erows.at[b], acc_e.at[sdix.at[b]], sems.at[b],
                             add=True)

        def wait_scatters(b):
            pltpu.make_async_copy(orows.at[b], acc_m.at[sdix.at[b]],
                                  sems.at[b]).wait()
            pltpu.make_async_copy(erows.at[b], acc_e.at[sdix.at[b]],
                                  sems.at[b]).wait()

        for b in range(NBUF - 1):
            issue_gathers(b, b)

        def ring_body(k4, carry):
            for b in range(NBUF):
                k = NBUF * k4 + b

                @pl.when(k + NBUF - 1 < nch)
                def _():
                    issue_gathers(k + NBUF - 1, (b + NBUF - 1) % NBUF)

                wait_gathers(b)

                @pl.when(k4 > 0)
                def _():
                    wait_scatters(b)

                pltpu.sync_copy(dst_hbm.at[pl.ds(woff + k * CH, CH)],
                                sdix.at[b])
                compute(b)
                issue_scatters(b)

            return carry

        lax.fori_loop(0, nch // NBUF, ring_body, 0)
        for b in range(NBUF):
            wait_scatters(b)
        plsc.subcore_barrier()
        pltpu.sync_copy(acc_m.at[pl.ds(s * STRIPE, STRIPE)],
                        m_out.at[c, pl.ds(s * STRIPE, STRIPE)])
        pltpu.sync_copy(acc_e.at[pl.ds(s * STRIPE, STRIPE)],
                        e_out.at[c, pl.ds(s * STRIPE, STRIPE)])

    return pl.kernel(
        body,
        out_type=[
            jax.ShapeDtypeStruct((NC, NPAD, HC), jnp.float32),
            jax.ShapeDtypeStruct((NC, NPAD, 16), jnp.float32),
        ],
        mesh=mesh,
        compiler_params=pltpu.CompilerParams(needs_layout_passes=False,
                                             use_tc_tiling_on_sc=False),
        scratch_types=[
            pltpu.VMEM((NBUF, CH), jnp.int32),
            pltpu.VMEM((NBUF, CH), jnp.int32),
            pltpu.VMEM((NBUF, CH), jnp.int32),
            pltpu.VMEM((NBUF, CH, HC), jnp.float32),
            pltpu.VMEM((NBUF, CH, 16), jnp.float32),
            pltpu.VMEM((NBUF, CH, 16), jnp.float32),
            pltpu.VMEM((NBUF, CH, HC), jnp.float32),
            pltpu.VMEM((NBUF, CH, 16), jnp.float32),
            pltpu.VMEM((16,), jnp.float32),
            pltpu.VMEM_SHARED((NPAD, HC), jnp.float32),
            pltpu.VMEM_SHARED((NPAD, 16), jnp.float32),
            pltpu.SemaphoreType.DMA((NBUF,)),
            pltpu.SemaphoreType.DMA((NBUF,)),
        ],
    )


import functools


@functools.lru_cache(maxsize=1)
def _get_edge_kernel():
    return _make_edge_kernel()


def _epi1_body(m_ref, e_ref, h_ref, as_ref, ad_ref, kv_ref, b_ref, r_ref,
               w2_ref, asm_ref, adm_ref, h2_ref, as2_ref, ad2_ref, mx_ref):
    i = pl.program_id(0)
    m = m_ref[...][0] + m_ref[...][1]
    e2 = e_ref[...][0] + e_ref[...][1]
    a = as_ref[...] + ad_ref[...]
    es = jnp.exp(_leaky(a) - kv_ref[...])
    den = jnp.dot(e2 + es, r_ref[...], preferred_element_type=jnp.float32)
    esb = jnp.dot(es, r_ref[...], preferred_element_type=jnp.float32)
    num = m + h_ref[...] * esb
    h1 = num / (den + 1e-16) + b_ref[...]
    h1e = jnp.where(h1 > 0, h1, jnp.exp(h1) - 1.0)
    h2 = jnp.dot(h1e, w2_ref[...], preferred_element_type=jnp.float32)
    a_s2 = jnp.dot(h2, asm_ref[...], preferred_element_type=jnp.float32)
    a_d2 = jnp.dot(h2, adm_ref[...], preferred_element_type=jnp.float32)
    h2_ref[...] = h2
    as2_ref[...] = a_s2
    ad2_ref[...] = a_d2

    @pl.when(i == 0)
    def _():
        mx_ref[...] = jnp.full((2, 16), -3.0e38, jnp.float32)

    upd = jnp.concatenate(
        [jnp.max(a_s2, axis=0, keepdims=True), jnp.max(a_d2, axis=0, keepdims=True)],
        axis=0,
    )
    mx_ref[...] = jnp.maximum(mx_ref[...], upd)


def _epi1(m1, e1, h1, as1, ad1, kv1, b1, r16, w2, asm2, adm2):
    return pl.pallas_call(
        _epi1_body,
        grid=(N // BM,),
        in_specs=[
            pl.BlockSpec((2, BM, HC), lambda i: (0, i, 0)),
            pl.BlockSpec((2, BM, 16), lambda i: (0, i, 0)),
            pl.BlockSpec((BM, HC), lambda i: (i, 0)),
            pl.BlockSpec((BM, 16), lambda i: (i, 0)),
            pl.BlockSpec((BM, 16), lambda i: (i, 0)),
            pl.BlockSpec((1, 16), lambda i: (0, 0)),
            pl.BlockSpec((1, HC), lambda i: (0, 0)),
            pl.BlockSpec((16, HC), lambda i: (0, 0)),
            pl.BlockSpec((HC, HC), lambda i: (0, 0)),
            pl.BlockSpec((HC, 16), lambda i: (0, 0)),
            pl.BlockSpec((HC, 16), lambda i: (0, 0)),
        ],
        out_specs=[
            pl.BlockSpec((BM, HC), lambda i: (i, 0)),
            pl.BlockSpec((BM, 16), lambda i: (i, 0)),
            pl.BlockSpec((BM, 16), lambda i: (i, 0)),
            pl.BlockSpec((2, 16), lambda i: (0, 0)),
        ],
        out_shape=[
            jax.ShapeDtypeStruct((N, HC), jnp.float32),
            jax.ShapeDtypeStruct((N, 16), jnp.float32),
            jax.ShapeDtypeStruct((N, 16), jnp.float32),
            jax.ShapeDtypeStruct((2, 16), jnp.float32),
        ],
    )(m1, e1, h1, as1, ad1, kv1, b1, r16, w2, asm2, adm2)


def _epi2_body(m_ref, e_ref, h_ref, as_ref, ad_ref, kv_ref, b_ref, r_ref,
               out_ref):
    m = m_ref[...][0] + m_ref[...][1]
    e2 = e_ref[...][0] + e_ref[...][1]
    a = as_ref[...] + ad_ref[...]
    es = jnp.exp(_leaky(a) - kv_ref[...])
    den = jnp.dot(e2 + es, r_ref[...], preferred_element_type=jnp.float32)
    esb = jnp.dot(es, r_ref[...], preferred_element_type=jnp.float32)
    num = m + h_ref[...] * esb
    o = num / (den + 1e-16) + b_ref[...]
    mx = jnp.max(o, axis=1, keepdims=True)
    z = o - mx
    lse = jnp.log(jnp.sum(jnp.exp(z), axis=1, keepdims=True))
    out_ref[...] = z - lse


def _epi2(m2, e2, h2, as2, ad2, kv2, b2, r16):
    return pl.pallas_call(
        _epi2_body,
        grid=(N // BM,),
        in_specs=[
            pl.BlockSpec((2, BM, HC), lambda i: (0, i, 0)),
            pl.BlockSpec((2, BM, 16), lambda i: (0, i, 0)),
            pl.BlockSpec((BM, HC), lambda i: (i, 0)),
            pl.BlockSpec((BM, 16), lambda i: (i, 0)),
            pl.BlockSpec((BM, 16), lambda i: (i, 0)),
            pl.BlockSpec((1, 16), lambda i: (0, 0)),
            pl.BlockSpec((1, HC), lambda i: (0, 0)),
            pl.BlockSpec((16, HC), lambda i: (0, 0)),
        ],
        out_specs=pl.BlockSpec((BM, HC), lambda i: (i, 0)),
        out_shape=jax.ShapeDtypeStruct((N, HC), jnp.float32),
    )(m2, e2, h2, as2, ad2, kv2, b2, r16)


_PERM = (jnp.arange(HC) % 8) * 8 + jnp.arange(HC) // 8


def _pack_mats1(att_src, att_dst):
    fs = att_src.reshape(HC)[_PERM]
    fd = att_dst.reshape(HC)[_PERM]
    rows = jnp.arange(HC)
    cols = jnp.arange(16)
    sel = ((cols[None, :] % 8) == (rows[:, None] % 8)).astype(jnp.float32)
    return fs[:, None] * sel, fd[:, None] * sel


def _pack_mats2(att_src, att_dst):
    fs = att_src.reshape(HC)
    fd = att_dst.reshape(HC)
    sel = jnp.ones((HC, 16), jnp.float32)
    return fs[:, None] * sel, fd[:, None] * sel


def _bcast_mat1():
    rows = jnp.arange(16)[:, None]
    t = jnp.arange(HC)[None, :]
    return (((t % 8) == rows) & (rows < 8)).astype(jnp.float32)


def _bcast_mat2():
    return (jnp.arange(16)[:, None] == 0).astype(jnp.float32) * jnp.ones(
        (1, HC), jnp.float32)


def kernel(x, edge_index, W1, att_src1, att_dst1, b1, W2, att_src2, att_dst2,
           b2):
    w1p = W1[:, _PERM]
    b1p = b1[_PERM]
    w2p = W2[_PERM, :]
    asm1, adm1 = _pack_mats1(att_src1, att_dst1)
    asm2, adm2 = _pack_mats2(att_src2, att_dst2)
    r16_1 = _bcast_mat1()
    r16_2 = _bcast_mat2()
    npad_e = EPAD - E
    srcp = jnp.concatenate([edge_index[0], jnp.zeros((npad_e,), jnp.int32)])
    dstp = jnp.concatenate([edge_index[1], jnp.full((npad_e,), N, jnp.int32)])
    z64 = jnp.zeros((NPAD, HC), jnp.float32)
    z16 = jnp.zeros((NPAD, 16), jnp.float32)
    b1r = b1p.reshape(1, HC)
    b2r = b2.reshape(1, HC)

    h1, as1, ad1, mx1 = _dense1(x, w1p, asm1, adm1)
    kv1 = _leaky(mx1[0] + mx1[1]).reshape(1, 16)
    m1, e1 = _get_edge_kernel()(h1, as1, ad1, srcp, dstp, kv1.reshape(16),
                                z64, z16)

    h2, as2, ad2, mx2 = _epi1(m1, e1, h1, as1, ad1, kv1, b1r, r16_1, w2p,
                              asm2, adm2)
    kv2 = _leaky(mx2[0] + mx2[1]).reshape(1, 16)
    m2, e2 = _get_edge_kernel()(h2, as2, ad2, srcp, dstp, kv2.reshape(16),
                                z64, z16)

    return _epi2(m2, e2, h2, as2, ad2, kv2, b2r, r16_2)
